# baseline (speedup 1.0000x reference)
import functools

import jax
import jax.numpy as jnp
from jax import lax
from jax.experimental import pallas as pl
from jax.experimental.pallas import tpu as pltpu, tpu_sc as plsc

B = 4096
N = 16384
NNZ = 12300
LANES = 16
NNZP = ((NNZ + LANES - 1) // LANES) * LANES
CHUNKS = NNZP // LANES

NUM_CORES = 2
NUM_SUBCORES = 16
NW = NUM_CORES * NUM_SUBCORES
ROWS_PER_W = B // NW
PAIRS_PER_W = ROWS_PER_W // 2


def _sc_kernel(x_hbm, w3_hbm, b_hbm, vel_hbm, i_hbm, j_hbm, out_hbm,
               iref, jref, vref, bias_v, x0, x1, o0, o1,
               sx0, sx1, so0, so1):
    wid = lax.axis_index("s") * NUM_CORES + lax.axis_index("c")
    base_row = wid * ROWS_PER_W

    pltpu.sync_copy(i_hbm, iref)
    pltpu.sync_copy(j_hbm, jref)
    pltpu.sync_copy(w3_hbm, vref)
    pltpu.sync_copy(vel_hbm, x0)
    pltpu.sync_copy(b_hbm, bias_v)

    def vals_body(c, carry):
        s = pl.ds(c * LANES, LANES)
        g = plsc.load_gather(x0, [jref[s]])
        vref[s] = vref[s] * g
        return carry

    lax.fori_loop(0, CHUNKS, vals_body, 0)

    xbufs, obufs = (x0, x1), (o0, o1)
    xsems, osems = (sx0, sx1), (so0, so1)

    pltpu.async_copy(x_hbm.at[base_row], x0, sx0)

    def pair_body(it, carry):
        for bslot in range(2):
            r = base_row + it * 2 + bslot
            xb, ob = xbufs[bslot], obufs[bslot]
            xs, os_ = xsems[bslot], osems[bslot]

            pltpu.make_async_copy(x_hbm.at[r], xb, xs).wait()

            @pl.when(it * 2 + bslot + 1 < ROWS_PER_W)
            def _():
                pltpu.async_copy(
                    x_hbm.at[r + 1], xbufs[1 - bslot], xsems[1 - bslot])

            @pl.when(it >= 1)
            def _():
                pltpu.make_async_copy(ob, out_hbm.at[r - 2], os_).wait()

            def bias_body(c, inner):
                s = pl.ds(c * LANES, LANES)
                ob[s] = bias_v[s]
                return inner

            lax.fori_loop(0, N // LANES, bias_body, 0, unroll=8)

            def chunk_body(c, inner):
                s = pl.ds(c * LANES, LANES)
                g = plsc.load_gather(xb, [jref[s]])
                plsc.addupdate_scatter(ob, [iref[s]], vref[s] * g)
                return inner

            lax.fori_loop(0, CHUNKS, chunk_body, 0, unroll=4)

            pltpu.async_copy(ob, out_hbm.at[r], os_)
        return carry

    lax.fori_loop(0, PAIRS_PER_W, pair_body, 0)

    pltpu.make_async_copy(o0, out_hbm.at[base_row + ROWS_PER_W - 2], so0).wait()
    pltpu.make_async_copy(o1, out_hbm.at[base_row + ROWS_PER_W - 1], so1).wait()


_mesh = plsc.VectorSubcoreMesh(core_axis_name="c", subcore_axis_name="s")

_call = functools.partial(
    pl.kernel,
    mesh=_mesh,
    out_type=jax.ShapeDtypeStruct((B, N), jnp.float32),
    compiler_params=pltpu.CompilerParams(needs_layout_passes=False),
    scratch_types=[
        pltpu.VMEM((NNZP,), jnp.int32),
        pltpu.VMEM((NNZP,), jnp.int32),
        pltpu.VMEM((NNZP,), jnp.float32),
        pltpu.VMEM((N,), jnp.float32),
        pltpu.VMEM((N,), jnp.float32),
        pltpu.VMEM((N,), jnp.float32),
        pltpu.VMEM((N,), jnp.float32),
        pltpu.VMEM((N,), jnp.float32),
        pltpu.SemaphoreType.DMA,
        pltpu.SemaphoreType.DMA,
        pltpu.SemaphoreType.DMA,
        pltpu.SemaphoreType.DMA,
    ],
)(_sc_kernel)


def kernel(inputs, W3, b, velocity, I, J):
    pad = NNZP - NNZ
    i_p = jnp.concatenate([I, jnp.zeros((pad,), jnp.int32)])
    j_p = jnp.concatenate([J, jnp.zeros((pad,), jnp.int32)])
    w_p = jnp.concatenate([W3, jnp.zeros((pad,), jnp.float32)])
    return _call(inputs, w_p, b, velocity, i_p, j_p)


# speedup vs baseline: 2.5029x; 2.5029x over previous
import functools

import jax
import jax.numpy as jnp
from jax import lax
from jax.experimental import pallas as pl
from jax.experimental.pallas import tpu as pltpu, tpu_sc as plsc

B = 4096
N = 16384
NNZ = 12300
LANES = 16
NNZP = ((NNZ + LANES - 1) // LANES) * LANES
CHUNKS = NNZP // LANES

NUM_CORES = 2
NUM_SUBCORES = 16
NW = NUM_CORES * NUM_SUBCORES
ROWS_PER_W = B // NW
PAIRS_PER_W = ROWS_PER_W // 2


def _sc_kernel(x_hbm, w3_hbm, b_hbm, vel_hbm, ji_hbm, out_hbm,
               jiref, vref, bias_v, x0, x1, o0, o1):
    wid = lax.axis_index("s") * NUM_CORES + lax.axis_index("c")
    base_row = wid * ROWS_PER_W

    pltpu.sync_copy(ji_hbm, jiref)
    pltpu.sync_copy(w3_hbm, vref)
    pltpu.sync_copy(vel_hbm, x0)
    pltpu.sync_copy(b_hbm, bias_v)

    @plsc.parallel_loop(0, CHUNKS, unroll=4)
    def _(c):
        s = pl.ds(c * LANES, LANES)
        j = jiref[s] & (N - 1)
        g = plsc.load_gather(x0, [j])
        vref[s] = vref[s] * g

    def pair_body(it, carry):
        r0 = base_row + it * 2
        pltpu.sync_copy(x_hbm.at[r0], x0)
        pltpu.sync_copy(x_hbm.at[r0 + 1], x1)

        @plsc.parallel_loop(0, N // LANES, unroll=8)
        def _(c):
            s = pl.ds(c * LANES, LANES)
            bv = bias_v[s]
            o0[s] = bv
            o1[s] = bv

        @plsc.parallel_loop(0, CHUNKS, unroll=4)
        def _(c):
            s = pl.ds(c * LANES, LANES)
            ji = jiref[s]
            v = vref[s]
            j = ji & (N - 1)
            i = lax.shift_right_logical(ji, 14)
            g0 = plsc.load_gather(x0, [j])
            plsc.addupdate_scatter(o0, [i], v * g0)
            g1 = plsc.load_gather(x1, [j])
            plsc.addupdate_scatter(o1, [i], v * g1)

        pltpu.sync_copy(o0, out_hbm.at[r0])
        pltpu.sync_copy(o1, out_hbm.at[r0 + 1])
        return carry

    lax.fori_loop(0, PAIRS_PER_W, pair_body, 0)


_mesh = plsc.VectorSubcoreMesh(core_axis_name="c", subcore_axis_name="s")

_call = functools.partial(
    pl.kernel,
    mesh=_mesh,
    out_type=jax.ShapeDtypeStruct((B, N), jnp.float32),
    compiler_params=pltpu.CompilerParams(needs_layout_passes=False),
    scratch_types=[
        pltpu.VMEM((NNZP,), jnp.int32),
        pltpu.VMEM((NNZP,), jnp.float32),
        pltpu.VMEM((N,), jnp.float32),
        pltpu.VMEM((N,), jnp.float32),
        pltpu.VMEM((N,), jnp.float32),
        pltpu.VMEM((N,), jnp.float32),
        pltpu.VMEM((N,), jnp.float32),
    ],
)(_sc_kernel)


def kernel(inputs, W3, b, velocity, I, J):
    pad = NNZP - NNZ
    ji = I * N + J
    ji_p = jnp.concatenate([ji, jnp.zeros((pad,), jnp.int32)])
    w_p = jnp.concatenate([W3, jnp.zeros((pad,), jnp.float32)])
    return _call(inputs, w_p, b, velocity, ji_p)


# speedup vs baseline: 3.5591x; 1.4220x over previous
import functools

import jax
import jax.numpy as jnp
from jax import lax
from jax.experimental import pallas as pl
from jax.experimental.pallas import tpu as pltpu, tpu_sc as plsc

B = 4096
N = 16384
NNZ = 12300
LANES = 16
NNZP = ((NNZ + LANES - 1) // LANES) * LANES
CHUNKS = NNZP // LANES

NUM_CORES = 2
NUM_SUBCORES = 16
NW = NUM_CORES * NUM_SUBCORES
ROWS_PER_W = B // NW
PAIRS_PER_W = ROWS_PER_W // 2


def _sc_kernel(x_hbm, w3_hbm, b_hbm, vel_hbm, ji_hbm, out_hbm,
               jiref, vref, bias_v, x0, x1, o0, o1,
               sx0, sx1, so0, so1):
    wid = lax.axis_index("s") * NUM_CORES + lax.axis_index("c")
    base_row = wid * ROWS_PER_W

    pltpu.sync_copy(ji_hbm, jiref)
    pltpu.sync_copy(w3_hbm, vref)
    pltpu.sync_copy(vel_hbm, x0)
    pltpu.sync_copy(b_hbm, bias_v)

    @plsc.parallel_loop(0, CHUNKS, unroll=4)
    def _(c):
        s = pl.ds(c * LANES, LANES)
        j = jiref[s] & (N - 1)
        g = plsc.load_gather(x0, [j])
        vref[s] = vref[s] * g

    xbufs, obufs = (x0, x1), (o0, o1)
    xsems, osems = (sx0, sx1), (so0, so1)

    pltpu.async_copy(x_hbm.at[base_row], x0, sx0)

    def pair_body(it, carry):
        for bslot in range(2):
            r = base_row + it * 2 + bslot
            xb, ob = xbufs[bslot], obufs[bslot]
            xs, os_ = xsems[bslot], osems[bslot]

            pltpu.make_async_copy(x_hbm.at[r], xb, xs).wait()

            @pl.when(it * 2 + bslot + 1 < ROWS_PER_W)
            def _():
                pltpu.async_copy(
                    x_hbm.at[r + 1], xbufs[1 - bslot], xsems[1 - bslot])

            @pl.when(it >= 1)
            def _():
                pltpu.make_async_copy(ob, out_hbm.at[r - 2], os_).wait()

            @plsc.parallel_loop(0, N // LANES, unroll=8)
            def _(c):
                s = pl.ds(c * LANES, LANES)
                ob[s] = bias_v[s]

            @plsc.parallel_loop(0, CHUNKS, unroll=4)
            def _(c):
                s = pl.ds(c * LANES, LANES)
                ji = jiref[s]
                v = vref[s]
                j = ji & (N - 1)
                i = lax.shift_right_logical(ji, 14)
                g = plsc.load_gather(xb, [j])
                plsc.addupdate_scatter(ob, [i], v * g)

            pltpu.async_copy(ob, out_hbm.at[r], os_)
        return carry

    lax.fori_loop(0, PAIRS_PER_W, pair_body, 0)

    pltpu.make_async_copy(o0, out_hbm.at[base_row + ROWS_PER_W - 2], so0).wait()
    pltpu.make_async_copy(o1, out_hbm.at[base_row + ROWS_PER_W - 1], so1).wait()


_mesh = plsc.VectorSubcoreMesh(core_axis_name="c", subcore_axis_name="s")

_call = functools.partial(
    pl.kernel,
    mesh=_mesh,
    out_type=jax.ShapeDtypeStruct((B, N), jnp.float32),
    compiler_params=pltpu.CompilerParams(needs_layout_passes=False),
    scratch_types=[
        pltpu.VMEM((NNZP,), jnp.int32),
        pltpu.VMEM((NNZP,), jnp.float32),
        pltpu.VMEM((N,), jnp.float32),
        pltpu.VMEM((N,), jnp.float32),
        pltpu.VMEM((N,), jnp.float32),
        pltpu.VMEM((N,), jnp.float32),
        pltpu.VMEM((N,), jnp.float32),
        pltpu.SemaphoreType.DMA,
        pltpu.SemaphoreType.DMA,
        pltpu.SemaphoreType.DMA,
        pltpu.SemaphoreType.DMA,
    ],
)(_sc_kernel)


def kernel(inputs, W3, b, velocity, I, J):
    pad = NNZP - NNZ
    ji = I * N + J
    ji_p = jnp.concatenate([ji, jnp.zeros((pad,), jnp.int32)])
    w_p = jnp.concatenate([W3, jnp.zeros((pad,), jnp.float32)])
    return _call(inputs, w_p, b, velocity, ji_p)
